# baseline (speedup 1.0000x reference)
import functools

import jax
import jax.numpy as jnp
from jax import lax
from jax.experimental import pallas as pl
from jax.experimental.pallas import tpu as pltpu
from jax.experimental.pallas import tpu_sc as plsc

_ROW = 32
_CHUNK = 128


@functools.cache
def _build(num_models: int, num_boxes: int, batch: int, dim: int):
  info = plsc.get_sparse_core_info()
  nc, ns = info.num_cores, info.num_subcores
  nw = nc * ns
  b_per_w = batch // nw
  n_chunks = b_per_w // _CHUNK
  mesh = plsc.VectorSubcoreMesh(core_axis_name="c", subcore_axis_name="s")

  @functools.partial(
      pl.kernel,
      mesh=mesh,
      out_type=jax.ShapeDtypeStruct(
          (num_models, nw, n_chunks, _CHUNK, _ROW), jnp.float32),
      scratch_types=[
          pltpu.VMEM((n_chunks, _CHUNK), jnp.int32),
          pltpu.VMEM((num_models, n_chunks, _CHUNK, _ROW), jnp.float32),
          pltpu.SemaphoreType.DMA,
      ],
      compiler_params=pltpu.CompilerParams(use_tc_tiling_on_sc=False),
  )
  def gather(ids_hbm, t0_hbm, t1_hbm, out_hbm, idx_v, rows_v, sem):
    wid = lax.axis_index("s") * nc + lax.axis_index("c")
    base = wid * b_per_w
    for j in range(n_chunks):
      pltpu.sync_copy(ids_hbm.at[pl.ds(base + j * _CHUNK, _CHUNK)],
                      idx_v.at[j])
    copies = []
    for m, table in enumerate((t0_hbm, t1_hbm)):
      for j in range(n_chunks):
        copies.append(
            pltpu.async_copy(table.at[idx_v.at[j]], rows_v.at[m, j], sem))
    for c in copies:
      c.wait()
    for m in range(num_models):
      pltpu.sync_copy(rows_v.at[m], out_hbm.at[m, wid])

  return gather


def kernel(ids, boxes):
  num_models, num_boxes, two, dim = boxes.shape
  batch = ids.shape[0]
  t0 = boxes[0].reshape(num_boxes, _ROW)
  t1 = boxes[1].reshape(num_boxes, _ROW)
  out = _build(num_models, num_boxes, batch, dim)(
      ids.astype(jnp.int32), t0, t1)
  return out.reshape(num_models, batch, two, dim)
